# Initial kernel scaffold; baseline (speedup 1.0000x reference)
import jax
import jax.numpy as jnp
from jax.experimental import pallas as pl


def kernel(x):
    raise NotImplementedError("write your pallas kernel here")


# speedup vs baseline: 192.4208x; 192.4208x over previous
import jax
import jax.numpy as jnp
from jax.experimental import pallas as pl
from jax.experimental.pallas import tpu as pltpu


def _med3(a, b, c):
    return jnp.maximum(jnp.minimum(a, b), jnp.minimum(jnp.maximum(a, b), c))


def _median3x3_kernel(x_ref, o_ref):
    x = x_ref[0]
    up = jnp.concatenate([x[1:2, :], x[:-1, :]], axis=0)
    dn = jnp.concatenate([x[1:, :], x[-2:-1, :]], axis=0)

    mn = jnp.minimum(up, x)
    mx = jnp.maximum(up, x)
    lo = jnp.minimum(mn, dn)
    hi = jnp.maximum(mx, dn)
    mid = jnp.maximum(mn, jnp.minimum(mx, dn))

    def lsh(v):
        return jnp.concatenate([v[:, 1:2], v[:, :-1]], axis=1)

    def rsh(v):
        return jnp.concatenate([v[:, 1:], v[:, -2:-1]], axis=1)

    a = jnp.maximum(jnp.maximum(lsh(lo), lo), rsh(lo))
    b = _med3(lsh(mid), mid, rsh(mid))
    c = jnp.minimum(jnp.minimum(lsh(hi), hi), rsh(hi))
    o_ref[0] = _med3(a, b, c)


def kernel(x):
    B, C, H, W = x.shape
    xf = x.reshape(B * C, H, W)
    out = pl.pallas_call(
        _median3x3_kernel,
        grid=(B * C,),
        in_specs=[pl.BlockSpec((1, H, W), lambda i: (i, 0, 0))],
        out_specs=pl.BlockSpec((1, H, W), lambda i: (i, 0, 0)),
        out_shape=jax.ShapeDtypeStruct((B * C, H, W), x.dtype),
        compiler_params=pltpu.CompilerParams(
            dimension_semantics=("parallel",),
        ),
    )(xf)
    return out.reshape(B, C, H, W)


# speedup vs baseline: 237.0464x; 1.2319x over previous
import jax
import jax.numpy as jnp
from jax.experimental import pallas as pl
from jax.experimental.pallas import tpu as pltpu


def _med3(a, b, c):
    return jnp.maximum(jnp.minimum(a, b), jnp.minimum(jnp.maximum(a, b), c))


def _median3x3_kernel(x_ref, o_ref):
    x = x_ref[0]
    lf = jnp.concatenate([x[:, 1:2], x[:, :-1]], axis=1)
    rt = jnp.concatenate([x[:, 1:], x[:, -2:-1]], axis=1)

    mn = jnp.minimum(lf, x)
    mx = jnp.maximum(lf, x)
    lo = jnp.minimum(mn, rt)
    hi = jnp.maximum(mx, rt)
    mid = jnp.maximum(mn, jnp.minimum(mx, rt))

    def up(v):
        return jnp.concatenate([v[1:2, :], v[:-1, :]], axis=0)

    def dn(v):
        return jnp.concatenate([v[1:, :], v[-2:-1, :]], axis=0)

    a = jnp.maximum(jnp.maximum(up(lo), lo), dn(lo))
    b = _med3(up(mid), mid, dn(mid))
    c = jnp.minimum(jnp.minimum(up(hi), hi), dn(hi))
    o_ref[0] = _med3(a, b, c)


def kernel(x):
    B, C, H, W = x.shape
    xf = x.reshape(B * C, H, W)
    out = pl.pallas_call(
        _median3x3_kernel,
        grid=(B * C,),
        in_specs=[pl.BlockSpec((1, H, W), lambda i: (i, 0, 0))],
        out_specs=pl.BlockSpec((1, H, W), lambda i: (i, 0, 0)),
        out_shape=jax.ShapeDtypeStruct((B * C, H, W), x.dtype),
        compiler_params=pltpu.CompilerParams(
            dimension_semantics=("parallel",),
        ),
    )(xf)
    return out.reshape(B, C, H, W)


# speedup vs baseline: 259.5033x; 1.0947x over previous
import jax
import jax.numpy as jnp
from jax.experimental import pallas as pl
from jax.experimental.pallas import tpu as pltpu


def _med3(a, b, c):
    return jnp.maximum(jnp.minimum(a, b), jnp.minimum(jnp.maximum(a, b), c))


def _median3x3_kernel(x_ref, o_ref):
    for img in range(x_ref.shape[0]):
        _median3x3_one(x_ref, o_ref, img)


def _median3x3_one(x_ref, o_ref, img):
    x = x_ref[img]
    lf = jnp.concatenate([x[:, 1:2], x[:, :-1]], axis=1)
    rt = jnp.concatenate([x[:, 1:], x[:, -2:-1]], axis=1)

    mn = jnp.minimum(lf, x)
    mx = jnp.maximum(lf, x)
    lo = jnp.minimum(mn, rt)
    hi = jnp.maximum(mx, rt)
    mid = jnp.maximum(mn, jnp.minimum(mx, rt))

    def up(v):
        return jnp.concatenate([v[1:2, :], v[:-1, :]], axis=0)

    def dn(v):
        return jnp.concatenate([v[1:, :], v[-2:-1, :]], axis=0)

    a = jnp.maximum(jnp.maximum(up(lo), lo), dn(lo))
    b = _med3(up(mid), mid, dn(mid))
    c = jnp.minimum(jnp.minimum(up(hi), hi), dn(hi))
    o_ref[img] = _med3(a, b, c)


_IMGS_PER_STEP = 2


def kernel(x):
    B, C, H, W = x.shape
    n = B * C
    xf = x.reshape(n, H, W)
    g = _IMGS_PER_STEP
    out = pl.pallas_call(
        _median3x3_kernel,
        grid=(n // g,),
        in_specs=[pl.BlockSpec((g, H, W), lambda i: (i, 0, 0))],
        out_specs=pl.BlockSpec((g, H, W), lambda i: (i, 0, 0)),
        out_shape=jax.ShapeDtypeStruct((n, H, W), x.dtype),
        compiler_params=pltpu.CompilerParams(
            dimension_semantics=("parallel",),
        ),
    )(xf)
    return out.reshape(B, C, H, W)


# speedup vs baseline: 263.7442x; 1.0163x over previous
import jax
import jax.numpy as jnp
from jax.experimental import pallas as pl
from jax.experimental.pallas import tpu as pltpu


def _med3(a, b, c):
    return jnp.maximum(jnp.minimum(a, b), jnp.minimum(jnp.maximum(a, b), c))


def _median3x3_kernel(x_ref, o_ref):
    for img in range(x_ref.shape[0]):
        _median3x3_one(x_ref, o_ref, img)


def _median3x3_one(x_ref, o_ref, img):
    x = x_ref[img]
    lf = jnp.concatenate([x[:, 1:2], x[:, :-1]], axis=1)
    rt = jnp.concatenate([x[:, 1:], x[:, -2:-1]], axis=1)

    mn = jnp.minimum(lf, x)
    mx = jnp.maximum(lf, x)
    lo = jnp.minimum(mn, rt)
    hi = jnp.maximum(mx, rt)
    mid = jnp.maximum(mn, jnp.minimum(mx, rt))

    def up(v):
        return jnp.concatenate([v[1:2, :], v[:-1, :]], axis=0)

    def dn(v):
        return jnp.concatenate([v[1:, :], v[-2:-1, :]], axis=0)

    a = jnp.maximum(jnp.maximum(up(lo), lo), dn(lo))
    b = _med3(up(mid), mid, dn(mid))
    c = jnp.minimum(jnp.minimum(up(hi), hi), dn(hi))
    o_ref[img] = _med3(a, b, c)


_IMGS_PER_STEP = 4


def kernel(x):
    B, C, H, W = x.shape
    n = B * C
    xf = x.reshape(n, H, W)
    g = _IMGS_PER_STEP
    out = pl.pallas_call(
        _median3x3_kernel,
        grid=(n // g,),
        in_specs=[pl.BlockSpec((g, H, W), lambda i: (i, 0, 0))],
        out_specs=pl.BlockSpec((g, H, W), lambda i: (i, 0, 0)),
        out_shape=jax.ShapeDtypeStruct((n, H, W), x.dtype),
        compiler_params=pltpu.CompilerParams(
            dimension_semantics=("parallel",),
        ),
    )(xf)
    return out.reshape(B, C, H, W)


# speedup vs baseline: 289.2989x; 1.0969x over previous
import jax
import jax.numpy as jnp
from jax.experimental import pallas as pl
from jax.experimental.pallas import tpu as pltpu


def _med3(a, b, c):
    return jnp.maximum(jnp.minimum(a, b), jnp.minimum(jnp.maximum(a, b), c))


def _hsort(xs):
    lf = jnp.concatenate([xs[:, 1:2], xs[:, :-1]], axis=1)
    rt = jnp.concatenate([xs[:, 1:], xs[:, -2:-1]], axis=1)
    mn = jnp.minimum(lf, xs)
    mx = jnp.maximum(lf, xs)
    lo = jnp.minimum(mn, rt)
    hi = jnp.maximum(mx, rt)
    mid = jnp.maximum(mn, jnp.minimum(mx, rt))
    return lo, mid, hi


_STRIP = 64


def _median3x3_kernel(x_ref, o_ref):
    H = x_ref.shape[1]
    ns = H // _STRIP
    for img in range(x_ref.shape[0]):
        cur = _hsort(x_ref[img, 0:_STRIP, :])
        prev_last = None
        for s in range(ns):
            r0 = s * _STRIP
            r1 = r0 + _STRIP
            nxt = _hsort(x_ref[img, r1:r1 + _STRIP, :]) if s < ns - 1 else None
            res = []
            for i in range(3):
                v = cur[i]
                top = v[1:2] if s == 0 else prev_last[i]
                bot = v[-2:-1] if s == ns - 1 else nxt[i][0:1]
                up = jnp.concatenate([top, v[:-1]], axis=0)
                dn = jnp.concatenate([v[1:], bot], axis=0)
                res.append((up, v, dn))
            (au, ac, ad), (bu, bc, bd), (cu, cc, cd) = res
            a = jnp.maximum(jnp.maximum(au, ac), ad)
            b = _med3(bu, bc, bd)
            c = jnp.minimum(jnp.minimum(cu, cc), cd)
            o_ref[img, r0:r1, :] = _med3(a, b, c)
            prev_last = tuple(v[-1:] for v in cur)
            cur = nxt


_IMGS_PER_STEP = 2


def kernel(x):
    B, C, H, W = x.shape
    n = B * C
    xf = x.reshape(n, H, W)
    g = _IMGS_PER_STEP
    out = pl.pallas_call(
        _median3x3_kernel,
        grid=(n // g,),
        in_specs=[pl.BlockSpec((g, H, W), lambda i: (i, 0, 0))],
        out_specs=pl.BlockSpec((g, H, W), lambda i: (i, 0, 0)),
        out_shape=jax.ShapeDtypeStruct((n, H, W), x.dtype),
        compiler_params=pltpu.CompilerParams(
            dimension_semantics=("parallel",),
        ),
    )(xf)
    return out.reshape(B, C, H, W)
